# baseline (speedup 1.0000x reference)
import jax
import jax.numpy as jnp
from jax import lax
from jax.experimental import pallas as pl
from jax.experimental.pallas import tpu as pltpu
from jax.experimental.pallas import tpu_sc as plsc

T = 1024
HD = 64
FD = 64
NF = 4
B = 8

NC = 2
NS = 16
L = 16
NW = NC * NS
RPW = T // NW
CHUNK = 128
NG = CHUNK // L

_NEG_BIG = -3.0e38


def _wh_body(h_ref, ww_ref, wb_ref, o_ref):
    o_ref[...] = lax.dot_general(
        h_ref[...], ww_ref[...],
        (((1,), (1,)), ((), ())),
        preferred_element_type=jnp.float32) + wb_ref[...]


def _s_body(a_ref, h_ref, o_ref):
    o_ref[...] = jnp.dot(a_ref[...], h_ref[...],
                         preferred_element_type=jnp.float32)


def _sc_body(f_h, wh_h, sb_h, ft_h, an_h, a_h,
             wh_v, sb_v, fbuf, ftbuf, anbuf, sig_v, att_v):
    cid = lax.axis_index("c")
    sid = lax.axis_index("s")
    wid = sid * NC + cid

    pltpu.sync_copy(wh_h, wh_v)
    pltpu.sync_copy(sb_h, sb_v)

    iota = lax.iota(jnp.int32, L)
    zeros16 = jnp.zeros((L,), jnp.float32)
    zidx = jnp.zeros((L,), jnp.int32)
    sbv = sb_v[0:2 * B]

    for g in range(T // L):
        att_v[g * L:(g + 1) * L] = zeros16

    def row_step(r, carry):
        i = r * NW + wid

        s = jnp.int32(0)
        e = jnp.int32(0)
        for b in range(B):
            sb = sbv[2 * b]
            eb = sbv[2 * b + 1]
            own = (sb <= i) & (i < eb) & ((eb - sb) > 1)
            s = jnp.where(own, sb, s)
            e = jnp.where(own, eb, e)

        st0 = (s >> 4) << 4
        nchunks = (e - st0 + CHUNK - 1) >> 7
        g_lo = st0 >> 4
        g_hi = (e + L - 1) >> 4

        def chunk_step(k, mk):
            mx, kc = mk
            wlo = st0 + k * CHUNK
            whi = jnp.minimum(e, wlo + CHUNK)
            st = pl.multiple_of(jnp.minimum(wlo, T - CHUNK), 16)
            lo = jnp.maximum(s, wlo)

            pltpu.sync_copy(f_h.at[i, pl.ds(st, CHUNK)], fbuf)
            pltpu.sync_copy(ft_h.at[i, pl.ds(st, CHUNK)], ftbuf)
            pltpu.sync_copy(an_h.at[i, pl.ds(st, CHUNK)], anbuf)

            for g in range(NG):
                lr = g * L
                sig = zeros16
                for jj in range(L):
                    jrow = st + lr + jj
                    p0 = fbuf[lr + jj, 0:16] * wh_v[jrow, 0:16]
                    p1 = fbuf[lr + jj, 16:32] * wh_v[jrow, 16:32]
                    p2 = fbuf[lr + jj, 32:48] * wh_v[jrow, 32:48]
                    p3 = fbuf[lr + jj, 48:64] * wh_v[jrow, 48:64]
                    sj = jnp.sum((p0 + p1) + (p2 + p3))
                    sig = jnp.where(iota == jj, sj, sig)

                jlan = (st + lr) + iota
                ang = anbuf[lr:lr + L]
                dist = plsc.load_gather(ftbuf, [lr + iota, zidx])
                msk = (ang < 0.0) | (dist > 10.0)
                valid = (jlan >= lo) & (jlan < whi)
                sigm = jnp.where((jlan == i) | msk, -1000.0, sig)
                sigm = jnp.where(valid, sigm, _NEG_BIG)
                old = sig_v[pl.ds(st + lr, L)]
                sig_v[pl.ds(st + lr, L)] = jnp.where(jlan >= wlo, sigm, old)
                mx = jnp.maximum(mx, jnp.max(sigm))
                kc = kc + jnp.sum(jnp.where(msk & valid, 1, 0))
            return (mx, kc)

        mx, kc = lax.fori_loop(0, nchunks, chunk_step,
                               (jnp.float32(_NEG_BIG), jnp.int32(0)))

        def exp_step(g2, lsum):
            sg = sig_v[pl.ds(g2 * L, L)]
            p = jnp.exp(sg - mx)
            att_v[pl.ds(g2 * L, L)] = p
            return lsum + jnp.sum(p)

        lsum = lax.fori_loop(g_lo, g_hi, exp_step, jnp.float32(0.0))

        kzero = kc == (e - s - 1)
        lvec = jnp.full((L,), lsum, jnp.float32)
        scale = jnp.where(kzero, zeros16, 1.0 / lvec)

        def scale_step(g2, _):
            att_v[pl.ds(g2 * L, L)] = att_v[pl.ds(g2 * L, L)] * scale
            return 0

        lax.fori_loop(g_lo, g_hi, scale_step, 0)

        pltpu.sync_copy(att_v, a_h.at[i])

        def zero_step(g2, _):
            att_v[pl.ds(g2 * L, L)] = zeros16
            return 0

        lax.fori_loop(g_lo, g_hi, zero_step, 0)
        return carry

    lax.fori_loop(0, RPW, row_step, 0)


def kernel(f, h, sub_batches, features, hor_bearings_MTX, W_w, W_b):
    wh = pl.pallas_call(
        _wh_body,
        out_shape=jax.ShapeDtypeStruct((T, FD), jnp.float32),
    )(h, W_w, W_b.reshape(1, FD))

    mesh = plsc.VectorSubcoreMesh(core_axis_name="c", subcore_axis_name="s")
    a = pl.kernel(
        _sc_body,
        out_type=jax.ShapeDtypeStruct((T, T), jnp.float32),
        mesh=mesh,
        compiler_params=pltpu.CompilerParams(use_tc_tiling_on_sc=False,
                                             needs_layout_passes=False),
        scratch_types=[
            pltpu.VMEM((T, FD), jnp.float32),
            pltpu.VMEM((2 * B,), jnp.int32),
            pltpu.VMEM((CHUNK, FD), jnp.float32),
            pltpu.VMEM((CHUNK, NF), jnp.float32),
            pltpu.VMEM((CHUNK,), jnp.float32),
            pltpu.VMEM((T,), jnp.float32),
            pltpu.VMEM((T,), jnp.float32),
        ],
    )(f, wh, sub_batches.astype(jnp.int32).reshape(2 * B),
      features, hor_bearings_MTX)

    return pl.pallas_call(
        _s_body,
        out_shape=jax.ShapeDtypeStruct((T, HD), jnp.float32),
    )(a, h)


# speedup vs baseline: 3.3144x; 3.3144x over previous
import jax
import jax.numpy as jnp
from jax import lax
from jax.experimental import pallas as pl
from jax.experimental.pallas import tpu as pltpu
from jax.experimental.pallas import tpu_sc as plsc

T = 1024
HD = 64
FD = 64
NF = 4
B = 8

NC = 2
NS = 16
L = 16
NW = NC * NS
RPW = T // NW
CHUNK = 128
NG = CHUNK // L

_NEG_BIG = -3.0e38


def _wh_body(h_ref, ww_ref, wb_ref, o_ref):
    o_ref[...] = lax.dot_general(
        h_ref[...], ww_ref[...],
        (((1,), (1,)), ((), ())),
        preferred_element_type=jnp.float32) + wb_ref[...]


def _s_body(a_ref, h_ref, o_ref):
    o_ref[...] = jnp.dot(a_ref[...], h_ref[...],
                         preferred_element_type=jnp.float32)


def _sc_body(f_h, wh_h, sb_h, ds_h, an_h, a_h,
             wh_v, sb_v, fbuf, angb, dstb, attb, sig_v):
    cid = lax.axis_index("c")
    sid = lax.axis_index("s")
    wid = sid * NC + cid

    pltpu.sync_copy(wh_h, wh_v)
    pltpu.sync_copy(sb_h, sb_v)

    iota = lax.iota(jnp.int32, L)
    zeros16 = jnp.zeros((L,), jnp.float32)
    sbv = sb_v[0:2 * B]

    def row_step(r, carry):
        i = r * NW + wid

        s = jnp.int32(0)
        e = jnp.int32(0)
        for b in range(B):
            sb = sbv[2 * b]
            eb = sbv[2 * b + 1]
            own = (sb <= i) & (i < eb) & ((eb - sb) > 1)
            s = jnp.where(own, sb, s)
            e = jnp.where(own, eb, e)

        st0 = (s >> 4) << 4
        nchunks = (e - st0 + CHUNK - 1) >> 7
        g_lo = st0 >> 4
        g_hi = (e + L - 1) >> 4

        pltpu.sync_copy(an_h.at[i, 0], angb)
        pltpu.sync_copy(ds_h.at[i, 0], dstb)

        for a8 in range(8):
            for gg in range(8):
                attb[a8, gg * L:(gg + 1) * L] = zeros16

        def chunk_step(k, mk):
            mx, kc = mk
            wlo = st0 + k * CHUNK
            whi = jnp.minimum(e, wlo + CHUNK)
            st = pl.multiple_of(jnp.minimum(wlo, T - CHUNK), 16)
            lo = jnp.maximum(s, wlo)

            pltpu.sync_copy(f_h.at[i, pl.ds(st, CHUNK)], fbuf)

            for g in range(NG):
                lr = g * L
                sig = zeros16
                for jj in range(L):
                    joff = (st + (lr + jj)) * FD
                    p0 = fbuf[lr + jj, 0:16] * wh_v[pl.ds(joff, 16)]
                    p1 = fbuf[lr + jj, 16:32] * wh_v[pl.ds(joff + 16, 16)]
                    p2 = fbuf[lr + jj, 32:48] * wh_v[pl.ds(joff + 32, 16)]
                    p3 = fbuf[lr + jj, 48:64] * wh_v[pl.ds(joff + 48, 16)]
                    sj = jnp.sum((p0 + p1) + (p2 + p3))
                    sig = jnp.where(iota == jj, sj, sig)

                j0 = st + lr
                jlan = j0 + iota
                arow = j0 >> 7
                acol = pl.multiple_of(j0 & 127, 16)
                ang = angb[arow, pl.ds(acol, L)]
                dist = dstb[arow, pl.ds(acol, L)]
                msk = (ang < 0.0) | (dist > 10.0)
                valid = (jlan >= lo) & (jlan < whi)
                sigm = jnp.where((jlan == i) | msk, -1000.0, sig)
                sigm = jnp.where(valid, sigm, _NEG_BIG)
                old = sig_v[pl.ds(j0, L)]
                sig_v[pl.ds(j0, L)] = jnp.where(jlan >= wlo, sigm, old)
                mx = jnp.maximum(mx, jnp.max(sigm))
                kc = kc + jnp.sum(jnp.where(msk & valid, 1, 0))
            return (mx, kc)

        mx, kc = lax.fori_loop(0, nchunks, chunk_step,
                               (jnp.float32(_NEG_BIG), jnp.int32(0)))

        def exp_step(g2, lsum):
            sg = sig_v[pl.ds(g2 * L, L)]
            p = jnp.exp(sg - mx)
            attb[g2 >> 3, pl.ds(pl.multiple_of((g2 & 7) * L, 16), L)] = p
            return lsum + jnp.sum(p)

        lsum = lax.fori_loop(g_lo, g_hi, exp_step, jnp.float32(0.0))

        kzero = kc == (e - s - 1)
        lvec = jnp.full((L,), lsum, jnp.float32)
        scale = jnp.where(kzero, zeros16, 1.0 / lvec)

        def scale_step(g2, _):
            a8 = g2 >> 3
            cc = pl.multiple_of((g2 & 7) * L, 16)
            attb[a8, pl.ds(cc, L)] = attb[a8, pl.ds(cc, L)] * scale
            return 0

        lax.fori_loop(g_lo, g_hi, scale_step, 0)

        pltpu.sync_copy(attb, a_h.at[i, 0])
        return carry

    lax.fori_loop(0, RPW, row_step, 0)


def kernel(f, h, sub_batches, features, hor_bearings_MTX, W_w, W_b):
    wh = pl.pallas_call(
        _wh_body,
        out_shape=jax.ShapeDtypeStruct((T, FD), jnp.float32),
    )(h, W_w, W_b.reshape(1, FD))

    dist4 = features[:, :, 0].reshape(T, 1, 8, 128)
    ang4 = hor_bearings_MTX.reshape(T, 1, 8, 128)

    mesh = plsc.VectorSubcoreMesh(core_axis_name="c", subcore_axis_name="s")
    a4 = pl.kernel(
        _sc_body,
        out_type=jax.ShapeDtypeStruct((T, 1, 8, 128), jnp.float32),
        mesh=mesh,
        compiler_params=pltpu.CompilerParams(needs_layout_passes=False),
        scratch_types=[
            pltpu.VMEM((T * FD,), jnp.float32),
            pltpu.VMEM((2 * B,), jnp.int32),
            pltpu.VMEM((CHUNK, FD), jnp.float32),
            pltpu.VMEM((8, 128), jnp.float32),
            pltpu.VMEM((8, 128), jnp.float32),
            pltpu.VMEM((8, 128), jnp.float32),
            pltpu.VMEM((T,), jnp.float32),
        ],
    )(f, wh.reshape(T * FD), sub_batches.astype(jnp.int32).reshape(2 * B),
      dist4, ang4)

    return pl.pallas_call(
        _s_body,
        out_shape=jax.ShapeDtypeStruct((T, HD), jnp.float32),
    )(a4.reshape(T, T), h)
